# Initial kernel scaffold; baseline (speedup 1.0000x reference)
import jax
import jax.numpy as jnp
from jax.experimental import pallas as pl


def kernel(sentence_repr, entity_span_indices, W, b):
    raise NotImplementedError("write your pallas kernel here")


# speedup vs baseline: 15.7741x; 15.7741x over previous
import functools

import jax
import jax.numpy as jnp
from jax import lax
from jax.experimental import pallas as pl
from jax.experimental.pallas import tpu as pltpu
from jax.experimental.pallas import tpu_sc as plsc

B, S, D, NS = 4, 2048, 1024, 128
MAXW = 64
NWORKERS = 32
SPANS_PER_W = (B * NS) // NWORKERS
WPB = NS // SPANS_PER_W


def _pool_body(sent_hbm, starts_hbm, ends_hbm, out_hbm,
               starts_v, ends_v, x_v, res_v):
    nc = 2
    wid = lax.axis_index("s") * nc + lax.axis_index("c")
    b = wid // WPB
    blk = (wid % WPB) * SPANS_PER_W

    pltpu.sync_copy(starts_hbm.at[b, pl.ds(blk, SPANS_PER_W)], starts_v)
    pltpu.sync_copy(ends_hbm.at[b, pl.ds(blk, SPANS_PER_W)], ends_v)
    pltpu.sync_copy(sent_hbm.at[b, pl.ds(0, MAXW), :], x_v)

    sv = starts_v[...]
    ev = ends_v[...]
    for i in range(SPANS_PER_W):
        s_i = sv[i]
        e_i = ev[i]
        w_v = jnp.full((16,), (e_i - s_i + 1).astype(jnp.float32))

        for g in range(8):
            def rbody(r, carry, g=g):
                sums, maxs = carry
                ns, nm = [], []
                for c in range(8):
                    x = x_v[r, pl.ds(g * 128 + c * 16, 16)]
                    ns.append(sums[c] + x)
                    nm.append(jnp.maximum(maxs[c], x))
                return tuple(ns), tuple(nm)

            init = (tuple(jnp.zeros((16,), jnp.float32) for _ in range(8)),
                    tuple(jnp.full((16,), -1e32, jnp.float32) for _ in range(8)))
            sums, maxs = lax.fori_loop(s_i, e_i + 1, rbody, init)
            for c in range(8):
                res_v[i, pl.ds(g * 128 + c * 16, 16)] = maxs[c]
                res_v[i, pl.ds(D + g * 128 + c * 16, 16)] = sums[c] / w_v

    pltpu.sync_copy(res_v, out_hbm.at[b, pl.ds(blk, SPANS_PER_W), :])


def _matmul_body(cat_ref, w_ref, b_ref, o_ref):
    o_ref[...] = lax.dot_general(
        cat_ref[...], w_ref[...],
        dimension_numbers=(((1,), (1,)), ((), ())),
        preferred_element_type=jnp.float32,
    ) + b_ref[...]


def kernel(sentence_repr, entity_span_indices, W, b):
    starts = entity_span_indices[..., 0].astype(jnp.int32)
    ends = entity_span_indices[..., 1].astype(jnp.int32)

    pool = functools.partial(
        pl.kernel,
        mesh=plsc.VectorSubcoreMesh(core_axis_name="c", subcore_axis_name="s"),
        out_type=jax.ShapeDtypeStruct((B, NS, 2 * D), jnp.float32),
        scratch_types=[
            pltpu.VMEM((SPANS_PER_W,), jnp.int32),
            pltpu.VMEM((SPANS_PER_W,), jnp.int32),
            pltpu.VMEM((MAXW, D), jnp.float32),
            pltpu.VMEM((SPANS_PER_W, 2 * D), jnp.float32),
        ],
    )(_pool_body)
    cat = pool(sentence_repr, starts, ends)

    out = pl.pallas_call(
        _matmul_body,
        out_shape=jax.ShapeDtypeStruct((B * NS, D), jnp.float32),
    )(cat.reshape(B * NS, 2 * D), W, b.reshape(1, D))
    return out.reshape(B, NS, D)


# speedup vs baseline: 22.7427x; 1.4418x over previous
import functools

import jax
import jax.numpy as jnp
from jax import lax
from jax.experimental import pallas as pl
from jax.experimental.pallas import tpu as pltpu
from jax.experimental.pallas import tpu_sc as plsc

B, S, D, NS = 4, 2048, 1024, 128
MAXW = 64
FSL = 128
NCHUNK = FSL // 16
WPB = D // FSL

_SIZES = [MAXW - (1 << k) + 1 for k in range(7)]
_OFFS = [sum(_SIZES[:k]) for k in range(7)]
_TROWS = sum(_SIZES)


def _pool_body(sent_hbm, starts_hbm, ends_hbm, out_hbm,
               sv_ref, ev_ref, tv_ref, pv_ref, rmax_ref, rmean_ref):
    nc = 2
    wid = lax.axis_index("s") * nc + lax.axis_index("c")
    b = wid // WPB
    fs = (wid % WPB) * FSL

    pltpu.sync_copy(starts_hbm.at[b], sv_ref.at[pl.ds(0, NS)])
    pltpu.sync_copy(ends_hbm.at[b], ev_ref.at[pl.ds(0, NS)])
    pltpu.sync_copy(sent_hbm.at[b, pl.ds(0, MAXW), pl.ds(fs, FSL)],
                    tv_ref.at[pl.ds(0, MAXW), :])

    zero = jnp.zeros((16,), jnp.float32)
    for c in range(NCHUNK):
        pv_ref[0, pl.ds(c * 16, 16)] = zero

    def pbody(r, accs):
        new = []
        for c in range(NCHUNK):
            a = accs[c] + tv_ref[r, pl.ds(c * 16, 16)]
            pv_ref[r + 1, pl.ds(c * 16, 16)] = a
            new.append(a)
        return tuple(new)

    lax.fori_loop(0, MAXW, pbody, tuple(zero for _ in range(NCHUNK)))

    for k in range(1, 7):
        prev_off, off, d = _OFFS[k - 1], _OFFS[k], 1 << (k - 1)

        def lbody(i, carry, prev_off=prev_off, off=off, d=d):
            for c in range(NCHUNK):
                lo = tv_ref[i + prev_off, pl.ds(c * 16, 16)]
                hi = tv_ref[i + prev_off + d, pl.ds(c * 16, 16)]
                tv_ref[i + off, pl.ds(c * 16, 16)] = jnp.maximum(lo, hi)
            return carry

        lax.fori_loop(0, _SIZES[k], lbody, 0)

    def qbody(i, carry):
        s_i = sv_ref[pl.ds(i, 16)][0]
        e_i = ev_ref[pl.ds(i, 16)][0]
        w = e_i - s_i + 1
        base = jnp.int32(0)
        pw = jnp.int32(1)
        for k in range(1, 7):
            cond = w >= (1 << k)
            base = jnp.where(cond, jnp.int32(_OFFS[k]), base)
            pw = jnp.where(cond, jnp.int32(1 << k), pw)
        r1 = s_i + base
        r2 = e_i + 1 - pw + base
        rv = 1.0 / jnp.full((16,), w.astype(jnp.float32))
        for c in range(NCHUNK):
            m = jnp.maximum(tv_ref[r1, pl.ds(c * 16, 16)],
                            tv_ref[r2, pl.ds(c * 16, 16)])
            rmax_ref[i, pl.ds(c * 16, 16)] = m
            sm = (pv_ref[e_i + 1, pl.ds(c * 16, 16)]
                  - pv_ref[s_i, pl.ds(c * 16, 16)])
            rmean_ref[i, pl.ds(c * 16, 16)] = sm * rv
        return carry

    lax.fori_loop(0, NS, qbody, 0)

    pltpu.sync_copy(rmax_ref, out_hbm.at[b, :, pl.ds(fs, FSL)])
    pltpu.sync_copy(rmean_ref, out_hbm.at[b, :, pl.ds(D + fs, FSL)])


def _matmul_body(cat_ref, w_ref, b_ref, o_ref):
    o_ref[...] = lax.dot_general(
        cat_ref[...].astype(jnp.bfloat16), w_ref[...].astype(jnp.bfloat16),
        dimension_numbers=(((1,), (1,)), ((), ())),
        preferred_element_type=jnp.float32,
    ) + b_ref[...]


def kernel(sentence_repr, entity_span_indices, W, b):
    starts = entity_span_indices[..., 0].astype(jnp.int32)
    ends = entity_span_indices[..., 1].astype(jnp.int32)

    pool = functools.partial(
        pl.kernel,
        mesh=plsc.VectorSubcoreMesh(core_axis_name="c", subcore_axis_name="s"),
        out_type=jax.ShapeDtypeStruct((B, NS, 2 * D), jnp.float32),
        scratch_types=[
            pltpu.VMEM((NS + 16,), jnp.int32),
            pltpu.VMEM((NS + 16,), jnp.int32),
            pltpu.VMEM((_TROWS, FSL), jnp.float32),
            pltpu.VMEM((MAXW + 1, FSL), jnp.float32),
            pltpu.VMEM((NS, FSL), jnp.float32),
            pltpu.VMEM((NS, FSL), jnp.float32),
        ],
    )(_pool_body)
    cat = pool(sentence_repr, starts, ends)

    out = pl.pallas_call(
        _matmul_body,
        out_shape=jax.ShapeDtypeStruct((B * NS, D), jnp.float32),
    )(cat.reshape(B * NS, 2 * D), W, b.reshape(1, D))
    return out.reshape(B, NS, D)


# speedup vs baseline: 29.3276x; 1.2895x over previous
import functools

import jax
import jax.numpy as jnp
from jax import lax
from jax.experimental import pallas as pl
from jax.experimental.pallas import tpu as pltpu
from jax.experimental.pallas import tpu_sc as plsc

B, S, D, NS = 4, 2048, 1024, 128
MAXW = 64
FSL = 128
NCHUNK = FSL // 16
WPB = D // FSL

_SIZES = [MAXW - (1 << k) + 1 for k in range(7)]
_OFFS = [sum(_SIZES[:k]) for k in range(7)]
_TROWS = sum(_SIZES)


def _pool_body(sent_hbm, esi_hbm, out_hbm,
               sev_ref, tv_ref, pv_ref, rmax_ref, rmean_ref):
    nc = 2
    wid = lax.axis_index("s") * nc + lax.axis_index("c")
    b = wid // WPB
    fs = (wid % WPB) * FSL

    pltpu.sync_copy(esi_hbm.at[b], sev_ref.at[pl.ds(0, 2 * NS)])
    pltpu.sync_copy(sent_hbm.at[b, pl.ds(0, MAXW), pl.ds(fs, FSL)],
                    tv_ref.at[pl.ds(0, MAXW), :])

    zero = jnp.zeros((16,), jnp.float32)
    for c in range(NCHUNK):
        pv_ref[0, pl.ds(c * 16, 16)] = zero

    @plsc.parallel_loop(0, MAXW, carry=tuple(zero for _ in range(NCHUNK)))
    def _(r, accs):
        new = []
        for c in range(NCHUNK):
            a = accs[c] + tv_ref[r, pl.ds(c * 16, 16)]
            pv_ref[r + 1, pl.ds(c * 16, 16)] = a
            new.append(a)
        return tuple(new)

    for k in range(1, 7):
        prev_off, off, d = _OFFS[k - 1], _OFFS[k], 1 << (k - 1)

        @plsc.parallel_loop(0, _SIZES[k])
        def _(i, prev_off=prev_off, off=off, d=d):
            for c in range(NCHUNK):
                lo = tv_ref[i + prev_off, pl.ds(c * 16, 16)]
                hi = tv_ref[i + prev_off + d, pl.ds(c * 16, 16)]
                tv_ref[i + off, pl.ds(c * 16, 16)] = jnp.maximum(lo, hi)

    @plsc.parallel_loop(0, NS, unroll=2)
    def _(i):
        se = sev_ref[pl.ds(2 * i, 16)]
        s_i = se[0]
        e_i = se[1]
        w = e_i - s_i + 1
        base = jnp.int32(0)
        pw = jnp.int32(1)
        for k in range(1, 7):
            cond = w >= (1 << k)
            base = jnp.where(cond, jnp.int32(_OFFS[k]), base)
            pw = jnp.where(cond, jnp.int32(1 << k), pw)
        r1 = s_i + base
        r2 = e_i + 1 - pw + base
        rv = 1.0 / jnp.full((16,), w.astype(jnp.float32))
        for c in range(NCHUNK):
            m = jnp.maximum(tv_ref[r1, pl.ds(c * 16, 16)],
                            tv_ref[r2, pl.ds(c * 16, 16)])
            rmax_ref[i, pl.ds(c * 16, 16)] = m
            sm = (pv_ref[e_i + 1, pl.ds(c * 16, 16)]
                  - pv_ref[s_i, pl.ds(c * 16, 16)])
            rmean_ref[i, pl.ds(c * 16, 16)] = sm * rv

    pltpu.sync_copy(rmax_ref, out_hbm.at[b, :, pl.ds(fs, FSL)])
    pltpu.sync_copy(rmean_ref, out_hbm.at[b, :, pl.ds(D + fs, FSL)])


def _matmul_body(cat_ref, w_ref, b_ref, o_ref):
    o_ref[...] = lax.dot_general(
        cat_ref[...].astype(jnp.bfloat16), w_ref[...].astype(jnp.bfloat16),
        dimension_numbers=(((1,), (1,)), ((), ())),
        preferred_element_type=jnp.float32,
    ) + b_ref[...]


def kernel(sentence_repr, entity_span_indices, W, b):
    esi = entity_span_indices.astype(jnp.int32).reshape(B, 2 * NS)

    pool = functools.partial(
        pl.kernel,
        mesh=plsc.VectorSubcoreMesh(core_axis_name="c", subcore_axis_name="s"),
        out_type=jax.ShapeDtypeStruct((B, NS, 2 * D), jnp.float32),
        scratch_types=[
            pltpu.VMEM((2 * NS + 16,), jnp.int32),
            pltpu.VMEM((_TROWS, FSL), jnp.float32),
            pltpu.VMEM((MAXW + 1, FSL), jnp.float32),
            pltpu.VMEM((NS, FSL), jnp.float32),
            pltpu.VMEM((NS, FSL), jnp.float32),
        ],
    )(_pool_body)
    cat = pool(sentence_repr, esi)

    out = pl.pallas_call(
        _matmul_body,
        out_shape=jax.ShapeDtypeStruct((B * NS, D), jnp.float32),
    )(cat.reshape(B * NS, 2 * D), W, b.reshape(1, D))
    return out.reshape(B, NS, D)
